# Initial kernel scaffold; baseline (speedup 1.0000x reference)
import jax
import jax.numpy as jnp
from jax.experimental import pallas as pl


def kernel(hidden_states, w_router, w_gate, w_up, w_down, ws_gate, ws_up, ws_down):
    raise NotImplementedError("write your pallas kernel here")


# speedup vs baseline: 2.9051x; 2.9051x over previous
import functools

import jax
import jax.numpy as jnp
from jax import lax
from jax.experimental import pallas as pl
from jax.experimental.pallas import tpu as pltpu
from jax.experimental.pallas import tpu_sc as plsc

E = 8
D = 1024
F = 2048
T = 4096
BT = 256
NB = T // BT + E
P = NB * BT

NC = 2
NS = 16
NW = NC * NS
TPW = T // NW
CH = 64

_f32 = jnp.float32
_i32 = jnp.int32


def _routing_body(x_ref, wr_ref, xsc_ref, p_ref, be_ref, nu_ref):
    x = x_ref[...]
    logits = jnp.dot(x, wr_ref[...], preferred_element_type=_f32)
    m = jnp.max(logits, axis=1, keepdims=True)
    eids = lax.broadcasted_iota(_i32, (T, E), 1)
    idx = jnp.min(jnp.where(logits == m, eids, E), axis=1, keepdims=True)
    score = jax.nn.sigmoid(m)
    xsc_ref[...] = x * score

    oh = (eids == idx).astype(_f32)
    tri = (lax.broadcasted_iota(_i32, (BT, BT), 0)
           >= lax.broadcasted_iota(_i32, (BT, BT), 1)).astype(_f32)
    carry = jnp.zeros((1, E), _f32)
    cums = []
    for i in range(T // BT):
        c = jnp.dot(tri, oh[i * BT:(i + 1) * BT, :],
                    preferred_element_type=_f32) + carry
        cums.append(c)
        carry = c[BT - 1:BT, :]
    cum = jnp.concatenate(cums, axis=0)
    counts = carry

    nb = (counts.astype(_i32) + (BT - 1)) // BT
    mstrict = (lax.broadcasted_iota(_i32, (E, E), 0)
               < lax.broadcasted_iota(_i32, (E, E), 1)).astype(_f32)
    bstart_f = jnp.dot(nb.astype(_f32), mstrict,
                       preferred_element_type=_f32)
    pstart_f = bstart_f * float(BT)
    pvals = jnp.sum(oh * (pstart_f + cum - 1.0), axis=1, keepdims=True)
    p_ref[...] = pvals.astype(_i32)

    b_iota = lax.broadcasted_iota(_i32, (NB, E), 0)
    e_iota = lax.broadcasted_iota(_i32, (NB, E), 1)
    ge = (b_iota >= bstart_f.astype(_i32)) & (e_iota >= 1)
    be_ref[...] = jnp.sum(ge.astype(_i32), axis=1, keepdims=True)
    nu_ref[...] = jnp.sum(nb, axis=1, keepdims=True)


def _routing_call(x, w_router):
    return pl.pallas_call(
        _routing_body,
        out_shape=[
            jax.ShapeDtypeStruct((T, D), _f32),
            jax.ShapeDtypeStruct((T, 1), _i32),
            jax.ShapeDtypeStruct((NB, 1), _i32),
            jax.ShapeDtypeStruct((1, 1), _i32),
        ],
    )(x, w_router)


def _dispatch_call(xsc, p):
    mesh = plsc.VectorSubcoreMesh(core_axis_name="c", subcore_axis_name="s",
                                  num_cores=NC, num_subcores=NS)

    @functools.partial(
        pl.kernel,
        out_type=jax.ShapeDtypeStruct((P, D), _f32),
        mesh=mesh,
        scratch_types=[
            pltpu.VMEM((CH,), _i32),
            pltpu.VMEM((CH,), _i32),
            pltpu.VMEM((CH, D), _f32),
            pltpu.SemaphoreType.DMA,
        ],
    )
    def scatter_k(xsc_hbm, p_hbm, out_hbm, idx0, idx1, rows, sem):
        wid = lax.axis_index("s") * NC + lax.axis_index("c")
        base = wid * TPW
        pltpu.sync_copy(p_hbm.at[pl.ds(base, CH)], idx0)
        pltpu.sync_copy(p_hbm.at[pl.ds(base + CH, CH)], idx1)
        pltpu.sync_copy(xsc_hbm.at[pl.ds(base, CH)], rows)
        pltpu.async_copy(rows, out_hbm.at[idx0], sem).wait()
        pltpu.sync_copy(xsc_hbm.at[pl.ds(base + CH, CH)], rows)
        pltpu.async_copy(rows, out_hbm.at[idx1], sem).wait()

    return scatter_k(xsc, p)


def _gmm_body(be_s, nu_s, xs_ref, w1_ref, w3_ref, w2_ref, y_ref):
    b = pl.program_id(0)

    @pl.when(b < nu_s[0])
    def _():
        x = xs_ref[...].astype(jnp.bfloat16)
        w1 = w1_ref[0].astype(jnp.bfloat16)
        w3 = w3_ref[0].astype(jnp.bfloat16)
        g = jnp.dot(x, w1, preferred_element_type=_f32)
        u = jnp.dot(x, w3, preferred_element_type=_f32)
        h = (g * jax.nn.sigmoid(g) * u).astype(jnp.bfloat16)
        w2 = w2_ref[0].astype(jnp.bfloat16)
        y_ref[...] = jnp.dot(h, w2, preferred_element_type=_f32)


def _gmm_call(be, nu, xs, w_gate, w_up, w_down):
    grid_spec = pltpu.PrefetchScalarGridSpec(
        num_scalar_prefetch=2,
        grid=(NB,),
        in_specs=[
            pl.BlockSpec((BT, D), lambda b, be, nu: (b, 0)),
            pl.BlockSpec((1, D, F), lambda b, be, nu: (be[b], 0, 0)),
            pl.BlockSpec((1, D, F), lambda b, be, nu: (be[b], 0, 0)),
            pl.BlockSpec((1, F, D), lambda b, be, nu: (be[b], 0, 0)),
        ],
        out_specs=pl.BlockSpec((BT, D), lambda b, be, nu: (b, 0)),
    )
    return pl.pallas_call(
        _gmm_body,
        grid_spec=grid_spec,
        out_shape=jax.ShapeDtypeStruct((P, D), _f32),
    )(be, nu, xs, w_gate, w_up, w_down)


def _combine_call(y, p):
    mesh = plsc.VectorSubcoreMesh(core_axis_name="c", subcore_axis_name="s",
                                  num_cores=NC, num_subcores=NS)

    @functools.partial(
        pl.kernel,
        out_type=jax.ShapeDtypeStruct((T, D), _f32),
        mesh=mesh,
        scratch_types=[
            pltpu.VMEM((CH,), _i32),
            pltpu.VMEM((CH,), _i32),
            pltpu.VMEM((CH, D), _f32),
            pltpu.SemaphoreType.DMA,
        ],
    )
    def gather_k(y_hbm, p_hbm, out_hbm, idx0, idx1, rows, sem):
        wid = lax.axis_index("s") * NC + lax.axis_index("c")
        base = wid * TPW
        pltpu.sync_copy(p_hbm.at[pl.ds(base, CH)], idx0)
        pltpu.sync_copy(p_hbm.at[pl.ds(base + CH, CH)], idx1)
        pltpu.async_copy(y_hbm.at[idx0], rows, sem).wait()
        pltpu.sync_copy(rows, out_hbm.at[pl.ds(base, CH)])
        pltpu.async_copy(y_hbm.at[idx1], rows, sem).wait()
        pltpu.sync_copy(rows, out_hbm.at[pl.ds(base + CH, CH)])

    return gather_k(y, p)


TB = 512


def _shared_body(x_ref, wg_ref, wu_ref, wd_ref, yc_ref, o_ref):
    x = x_ref[...].astype(jnp.bfloat16)
    g = jnp.dot(x, wg_ref[...].astype(jnp.bfloat16),
                preferred_element_type=_f32)
    u = jnp.dot(x, wu_ref[...].astype(jnp.bfloat16),
                preferred_element_type=_f32)
    h = (g * jax.nn.sigmoid(g) * u).astype(jnp.bfloat16)
    o_ref[...] = yc_ref[...] + jnp.dot(h, wd_ref[...].astype(jnp.bfloat16),
                                       preferred_element_type=_f32)


def _shared_call(x, ws_gate, ws_up, ws_down, yc):
    return pl.pallas_call(
        _shared_body,
        grid=(T // TB,),
        in_specs=[
            pl.BlockSpec((TB, D), lambda t: (t, 0)),
            pl.BlockSpec((D, F), lambda t: (0, 0)),
            pl.BlockSpec((D, F), lambda t: (0, 0)),
            pl.BlockSpec((F, D), lambda t: (0, 0)),
            pl.BlockSpec((TB, D), lambda t: (t, 0)),
        ],
        out_specs=pl.BlockSpec((TB, D), lambda t: (t, 0)),
        out_shape=jax.ShapeDtypeStruct((T, D), _f32),
    )(x, ws_gate, ws_up, ws_down, yc)


def kernel(hidden_states, w_router, w_gate, w_up, w_down,
           ws_gate, ws_up, ws_down):
    xsc, p2, be2, nu2 = _routing_call(hidden_states, w_router)
    p = p2.reshape(T)
    be = be2.reshape(NB)
    nu = nu2.reshape(1)
    xs = _dispatch_call(xsc, p)
    y = _gmm_call(be, nu, xs, w_gate, w_up, w_down)
    yc = _combine_call(y, p)
    return _shared_call(hidden_states, ws_gate, ws_up, ws_down, yc)
